# Initial kernel scaffold; baseline (speedup 1.0000x reference)
import jax
import jax.numpy as jnp
from jax.experimental import pallas as pl


def kernel(x, edge_index, W1_l, W1_r, b1, W2_l, W2_r, b2):
    raise NotImplementedError("write your pallas kernel here")


# speedup vs baseline: 5.5768x; 5.5768x over previous
import jax
import jax.numpy as jnp
from jax import lax
from jax.experimental import pallas as pl
from jax.experimental.pallas import tpu as pltpu
from jax.experimental.pallas import tpu_sc as plsc

N = 10000
E = 320000
D = 128
NC = 2
NS = 16
NW = NC * NS
EPW = E // NW
CHUNK = 80
NCHUNK = EPW // CHUNK
RPT = 624
REM = N - NS * RPT
BLK = 1000
NBLK = N // BLK
NPAD = 10240

_f32 = jnp.float32

_mesh = plsc.VectorSubcoreMesh(
    core_axis_name="c", subcore_axis_name="s", num_cores=NC, num_subcores=NS)
_sc_params = pltpu.CompilerParams(needs_layout_passes=False)


def _make_sc_agg(with_deg):
  def body(table, zagg, src, dst, *rest):
    if with_deg:
      agg_out, deg_out, src_v, dst_v, rows_v, hist_v, sem, agg_sh = rest
    else:
      agg_out, src_v, dst_v, rows_v, sem, agg_sh = rest
    c = lax.axis_index("c")
    s = lax.axis_index("s")
    wid = c * NS + s
    ebase = wid * EPW

    rbase = s * RPT
    pltpu.sync_copy(zagg.at[pl.ds(rbase, RPT)], agg_sh.at[pl.ds(rbase, RPT)])

    @pl.when(s == NS - 1)
    def _zero_rem():
      eb = NS * RPT
      pltpu.sync_copy(zagg.at[pl.ds(eb, REM)], agg_sh.at[pl.ds(eb, REM)])

    if with_deg:
      def zero_hist(r, carry):
        hist_v[pl.ds(r * 16, 16)] = jnp.zeros((16,), _f32)
        return carry

      lax.fori_loop(0, NPAD // 16, zero_hist, 0)

    plsc.subcore_barrier()

    ones16 = jnp.ones((16,), _f32)

    def step(j, carry):
      pltpu.sync_copy(src.at[pl.ds(ebase + j * CHUNK, CHUNK)], src_v)
      pltpu.sync_copy(dst.at[pl.ds(ebase + j * CHUNK, CHUNK)], dst_v)
      pltpu.async_copy(table.at[src_v], rows_v, sem).wait()
      pltpu.sync_copy(rows_v, agg_sh.at[dst_v], add=True)
      if with_deg:
        def count(g, carry2):
          idx = dst_v[pl.ds(g * 16, 16)]
          plsc.addupdate_scatter(hist_v, [idx], ones16)
          return carry2

        lax.fori_loop(0, CHUNK // 16, count, 0)
      return carry

    lax.fori_loop(0, NCHUNK, step, 0)

    plsc.subcore_barrier()

    obase = c * N + rbase
    pltpu.sync_copy(agg_sh.at[pl.ds(rbase, RPT)], agg_out.at[pl.ds(obase, RPT)])

    @pl.when(s == NS - 1)
    def _write_rem():
      eb = NS * RPT
      pltpu.sync_copy(agg_sh.at[pl.ds(eb, REM)],
                      agg_out.at[pl.ds(c * N + eb, REM)])

    if with_deg:
      pltpu.sync_copy(hist_v, deg_out.at[wid])

  out_type = [jax.ShapeDtypeStruct((NC * N, D), _f32)]
  scratch = [
      pltpu.VMEM((CHUNK,), jnp.int32),
      pltpu.VMEM((CHUNK,), jnp.int32),
      pltpu.VMEM((CHUNK, D), _f32),
  ]
  if with_deg:
    out_type.append(jax.ShapeDtypeStruct((NW, NPAD), _f32))
    scratch.append(pltpu.VMEM((NPAD,), _f32))
  scratch += [pltpu.SemaphoreType.DMA, pltpu.VMEM_SHARED((N, D), _f32)]

  return pl.kernel(
      body,
      out_type=tuple(out_type) if with_deg else out_type[0],
      mesh=_mesh,
      compiler_params=_sc_params,
      scratch_types=scratch,
  )


_sc_agg_deg = _make_sc_agg(True)
_sc_agg = _make_sc_agg(False)


def _tc_deginv_body(dp, o):
  ones = jnp.ones((NW, 1), _f32)
  deg = lax.dot_general(dp[...], ones, (((0,), (0,)), ((), ())),
                        preferred_element_type=_f32)
  o[...] = 1.0 / jnp.maximum(deg, 1.0)


_tc_deginv = pl.pallas_call(
    _tc_deginv_body,
    in_specs=[pl.BlockSpec((NW, NPAD), lambda: (0, 0))],
    out_specs=pl.BlockSpec((NPAD, 1), lambda: (0, 0)),
    out_shape=jax.ShapeDtypeStruct((NPAD, 1), _f32),
)


def _tc_layer(relu):
  def body(a0, a1, inv_ref, x, wl, wr, b, o):
    mean = (a0[...] + a1[...]) * inv_ref[...]
    acc = (jnp.dot(mean, wl[...], preferred_element_type=_f32)
           + jnp.dot(x[...], wr[...], preferred_element_type=_f32)
           + b[...])
    o[...] = jnp.maximum(acc, 0.0) if relu else acc

  return pl.pallas_call(
      body,
      grid=(NBLK,),
      in_specs=[
          pl.BlockSpec((BLK, D), lambda i: (i, 0)),
          pl.BlockSpec((BLK, D), lambda i: (i + NBLK, 0)),
          pl.BlockSpec((BLK, 1), lambda i: (i, 0)),
          pl.BlockSpec((BLK, D), lambda i: (i, 0)),
          pl.BlockSpec((D, D), lambda i: (0, 0)),
          pl.BlockSpec((D, D), lambda i: (0, 0)),
          pl.BlockSpec((1, D), lambda i: (0, 0)),
      ],
      out_specs=pl.BlockSpec((BLK, D), lambda i: (i, 0)),
      out_shape=jax.ShapeDtypeStruct((N, D), _f32),
  )


_tc_relu = _tc_layer(True)
_tc_lin = _tc_layer(False)


def kernel(x, edge_index, W1_l, W1_r, b1, W2_l, W2_r, b2):
  src = edge_index[0]
  dst = edge_index[1]
  zagg = jnp.zeros((N, D), _f32)
  agg1, degp = _sc_agg_deg(x, zagg, src, dst)
  inv = _tc_deginv(degp)
  h = _tc_relu(agg1, agg1, inv, x, W1_l, W1_r, b1.reshape(1, D))
  agg2 = _sc_agg(h, zagg, src, dst)
  out = _tc_lin(agg2, agg2, inv, h, W2_l, W2_r, b2.reshape(1, D))
  return out
